# initial kernel scaffold (unmeasured)
import functools

import jax
import jax.numpy as jnp
from jax import lax
from jax.experimental import pallas as pl
from jax.experimental.pallas import tpu as pltpu

N_DEV = 8
SQ = 1024
SKV_LOC = 1024
HQ_LOC = 8
DH = 128
NG = 32
BAND_KV = 1152
SCALE = 0.08838834764831843

_sig = getattr(pl, "semaphore_signal", None) or pltpu.semaphore_signal
_swait = getattr(pl, "semaphore_wait", None) or pltpu.semaphore_wait

MESH = pl.DeviceIdType.MESH


def kernel(x, Wq, K_ext, V_ext, Wo):
    def body(x_ref, wq_ref, k_ref, v_ref, wo_ref, out_ref,
             q_sc, qg_all, kc_sc, vc_sc, mine_nd, s_nd, comm_nd,
             kb_sc, vb_sc, partial_sc, comm_out,
             qg_send, qg_recv, scat_send, kbvb_recv, loc_sems, fill_sems,
             ring_send, ring_recv, credit_sem, minibar_sem):
        pos = lax.axis_index("i")
        left = lax.rem(pos + N_DEV - 1, N_DEV)
        right = lax.rem(pos + 1, N_DEV)

        barrier_sem = pltpu.get_barrier_semaphore()
        for off in range(1, N_DEV):
            _sig(barrier_sem, inc=1, device_id=(lax.rem(pos + off, N_DEV),),
                 device_id_type=MESH)
        _swait(barrier_sem, N_DEV - 1)

        def scatter_rdmas(src_pos):
            rdmas = []
            if src_pos == 0:
                ksl = lambda j: k_ref.at[0, :, 8 * j:8 * j + 8, :]
                vsl = lambda j: v_ref.at[0, :, 8 * j:8 * j + 8, :]
                kdst = kb_sc.at[0:1024]
                vdst = vb_sc.at[0:1024]
                ksem, vsem = 0, 2
            else:
                ksl = lambda j: k_ref.at[0, 0:128, 8 * j:8 * j + 8, :]
                vsl = lambda j: v_ref.at[0, 0:128, 8 * j:8 * j + 8, :]
                kdst = kb_sc.at[1024:1152]
                vdst = vb_sc.at[1024:1152]
                ksem, vsem = 1, 3
            s = 0
            for j in range(N_DEV):
                if j == src_pos:
                    continue
                rdmas.append(pltpu.make_async_remote_copy(
                    src_ref=ksl(j), dst_ref=kdst,
                    send_sem=scat_send.at[s], recv_sem=kbvb_recv.at[ksem],
                    device_id=(j,), device_id_type=MESH))
                rdmas.append(pltpu.make_async_remote_copy(
                    src_ref=vsl(j), dst_ref=vdst,
                    send_sem=scat_send.at[s + 7], recv_sem=kbvb_recv.at[vsem],
                    device_id=(j,), device_id_type=MESH))
                s += 1
            return rdmas

        def ring_allreduce(mine_ref, s_ref, comm_ref):
            for h in range(N_DEV - 1):
                slot = h % 2
                rdma = pltpu.make_async_remote_copy(
                    src_ref=s_ref, dst_ref=comm_ref.at[slot],
                    send_sem=ring_send.at[slot], recv_sem=ring_recv.at[slot],
                    device_id=(right,), device_id_type=MESH)
                if h >= 2:
                    _swait(credit_sem, 1)
                rdma.start()
                rdma.wait_send()
                rdma.wait_recv()
                s_ref[...] = comm_ref[slot] + mine_ref[...]
                if h < N_DEV - 3:
                    _sig(credit_sem, inc=1, device_id=(left,),
                         device_id_type=MESH)

        def neighbor_barrier():
            _sig(minibar_sem, inc=1, device_id=(left,), device_id_type=MESH)
            _sig(minibar_sem, inc=1, device_id=(right,), device_id_type=MESH)
            _swait(minibar_sem, 2)

        @pl.when(pos == 0)
        def _():
            for r in scatter_rdmas(0):
                r.start()
            pltpu.make_async_copy(k_ref.at[0, :, 0:8, :], kb_sc.at[0:1024],
                                  fill_sems.at[0]).start()
            pltpu.make_async_copy(v_ref.at[0, :, 0:8, :], vb_sc.at[0:1024],
                                  fill_sems.at[1]).start()

        @pl.when(pos == 1)
        def _():
            for r in scatter_rdmas(1):
                r.start()
            pltpu.make_async_copy(k_ref.at[0, 0:128, 8:16, :],
                                  kb_sc.at[1024:1152], fill_sems.at[0]).start()
            pltpu.make_async_copy(v_ref.at[0, 0:128, 8:16, :],
                                  vb_sc.at[1024:1152], fill_sems.at[1]).start()

        q = jnp.dot(x_ref[0], wq_ref[...], preferred_element_type=jnp.float32)
        q_sc[...] = q
        qg_all[pl.ds(pos, 1)] = q[0:32, :][None]

        qg_rdmas = []
        for off in range(1, N_DEV):
            tgt = lax.rem(pos + off, N_DEV)
            r = pltpu.make_async_remote_copy(
                src_ref=qg_all.at[pl.ds(pos, 1)],
                dst_ref=qg_all.at[pl.ds(pos, 1)],
                send_sem=qg_send.at[off], recv_sem=qg_recv.at[off],
                device_id=(tgt,), device_id_type=MESH)
            r.start()
            qg_rdmas.append(r)
        for offb in range(1, N_DEV):
            src = lax.rem(pos + N_DEV - offb, N_DEV)
            pltpu.make_async_remote_copy(
                src_ref=qg_all.at[pl.ds(pos, 1)],
                dst_ref=qg_all.at[pl.ds(src, 1)],
                send_sem=qg_send.at[offb], recv_sem=qg_recv.at[offb],
                device_id=(src,), device_id_type=MESH).wait_recv()

        qga = qg_all[...]
        qgt = qga.reshape(N_DEV, NG, HQ_LOC, DH)
        qgt = qgt.transpose(0, 2, 1, 3).reshape(64, NG, DH)

        num = jnp.zeros((64, NG, DH), jnp.float32)
        den = jnp.zeros((64, NG), jnp.float32)
        n_chunks = 8
        ck = SKV_LOC // n_chunks
        for c in range(n_chunks):
            pltpu.make_async_copy(k_ref.at[0, pl.ds(c * ck, ck), :, :],
                                  kc_sc, loc_sems.at[0]).start()
            pltpu.make_async_copy(v_ref.at[0, pl.ds(c * ck, ck), :, :],
                                  vc_sc, loc_sems.at[1]).start()
            pltpu.make_async_copy(k_ref.at[0, pl.ds(c * ck, ck), :, :],
                                  kc_sc, loc_sems.at[0]).wait()
            pltpu.make_async_copy(v_ref.at[0, pl.ds(c * ck, ck), :, :],
                                  vc_sc, loc_sems.at[1]).wait()
            kc = kc_sc[...].transpose(1, 0, 2)
            vc = vc_sc[...].transpose(1, 0, 2)
            sc = jax.lax.dot_general(
                qgt, kc, (((2,), (2,)), ((0,), (0,))),
                preferred_element_type=jnp.float32) * SCALE
            w = jnp.exp(sc)
            num = num + jax.lax.dot_general(
                w, vc, (((2,), (1,)), ((0,), (0,))),
                preferred_element_type=jnp.float32)
            den = den + jnp.sum(w, axis=2)

        den_pad = jnp.pad(den[:, None, :], ((0, 0), (0, 0), (0, DH - NG)))
        packed = jnp.concatenate([num, den_pad], axis=1)
        mine_nd[...] = packed
        s_nd[...] = packed

        for r in qg_rdmas:
            r.wait_send()

        ring_allreduce(mine_nd, s_nd, comm_nd)

        myslice = s_nd[pl.ds(pos * HQ_LOC, HQ_LOC)]
        g_num = myslice[:, 0:NG, :]
        g_den = myslice[:, NG, 0:NG]

        @pl.when(pos == 0)
        def _():
            for r in scatter_rdmas(0):
                r.wait_send()
            pltpu.make_async_copy(k_ref.at[0, :, 0:8, :], kb_sc.at[0:1024],
                                  fill_sems.at[0]).wait()
            pltpu.make_async_copy(v_ref.at[0, :, 0:8, :], vb_sc.at[0:1024],
                                  fill_sems.at[1]).wait()

        @pl.when(pos == 1)
        def _():
            for r in scatter_rdmas(1):
                r.wait_send()
            pltpu.make_async_copy(k_ref.at[0, 0:128, 8:16, :],
                                  kb_sc.at[1024:1152], fill_sems.at[0]).wait()
            pltpu.make_async_copy(v_ref.at[0, 0:128, 8:16, :],
                                  vb_sc.at[1024:1152], fill_sems.at[1]).wait()

        def recv_band(sem_i, dst, nrows):
            pltpu.make_async_remote_copy(
                src_ref=dst, dst_ref=dst,
                send_sem=scat_send.at[14], recv_sem=kbvb_recv.at[sem_i],
                device_id=(0,), device_id_type=MESH).wait_recv()

        @pl.when(pos != 0)
        def _():
            recv_band(0, kb_sc.at[0:1024], 1024)
            recv_band(2, vb_sc.at[0:1024], 1024)

        @pl.when(pos != 1)
        def _():
            recv_band(1, kb_sc.at[1024:1152], 128)
            recv_band(3, vb_sc.at[1024:1152], 128)

        qi = lax.broadcasted_iota(jnp.int32, (SQ - NG, BAND_KV), 0) + NG
        ki = lax.broadcasted_iota(jnp.int32, (SQ - NG, BAND_KV), 1)
        mask = (jnp.abs(qi - ki) <= 128) | (ki < 32)

        acc = jnp.zeros((SQ, 1024), jnp.float32)
        for hh in range(HQ_LOC):
            qb = q_sc[NG:, hh * DH:(hh + 1) * DH]
            kbh = kb_sc[:, hh, :]
            vbh = vb_sc[:, hh, :]
            sc = jax.lax.dot_general(
                qb, kbh, (((1,), (1,)), ((), ())),
                preferred_element_type=jnp.float32) * SCALE
            sc = jnp.where(mask, sc, -1e9)
            mx = jnp.max(sc, axis=1, keepdims=True)
            w = jnp.exp(sc - mx)
            w = w / jnp.sum(w, axis=1, keepdims=True)
            ctx_band = jax.lax.dot_general(
                w, vbh, (((1,), (0,)), ((), ())),
                preferred_element_type=jnp.float32)
            ctx_g = g_num[hh] / g_den[hh][:, None]
            ctx_h = jnp.concatenate([ctx_g, ctx_band], axis=0)
            acc = acc + jnp.dot(ctx_h, wo_ref[hh * DH:(hh + 1) * DH, :],
                                preferred_element_type=jnp.float32)
        partial_sc[...] = acc

        for r in range(2):
            neighbor_barrier()
            rows = pl.ds(r * 512, 512)
            out_ref[0, rows, :] = partial_sc[rows, :]
            ring_allreduce(partial_sc.at[rows], out_ref.at[0, rows, :],
                           comm_out)

    out_shape = jax.ShapeDtypeStruct((1, SQ, 1024), jnp.float32)
    grid_spec = pltpu.PrefetchScalarGridSpec(
        num_scalar_prefetch=0,
        in_specs=[
            pl.BlockSpec(memory_space=pltpu.VMEM),
            pl.BlockSpec(memory_space=pltpu.VMEM),
            pl.BlockSpec(memory_space=pltpu.ANY),
            pl.BlockSpec(memory_space=pltpu.ANY),
            pl.BlockSpec(memory_space=pltpu.VMEM),
        ],
        out_specs=pl.BlockSpec(memory_space=pltpu.VMEM),
        scratch_shapes=[
            pltpu.VMEM((SQ, 1024), jnp.float32),
            pltpu.VMEM((N_DEV, NG, 1024), jnp.float32),
            pltpu.VMEM((128, 64, DH), jnp.float32),
            pltpu.VMEM((128, 64, DH), jnp.float32),
            pltpu.VMEM((64, NG + 1, DH), jnp.float32),
            pltpu.VMEM((64, NG + 1, DH), jnp.float32),
            pltpu.VMEM((2, 64, NG + 1, DH), jnp.float32),
            pltpu.VMEM((BAND_KV, HQ_LOC, DH), jnp.float32),
            pltpu.VMEM((BAND_KV, HQ_LOC, DH), jnp.float32),
            pltpu.VMEM((SQ, 1024), jnp.float32),
            pltpu.VMEM((2, 512, 1024), jnp.float32),
            pltpu.SemaphoreType.DMA((N_DEV,)),
            pltpu.SemaphoreType.DMA((N_DEV,)),
            pltpu.SemaphoreType.DMA((16,)),
            pltpu.SemaphoreType.DMA((4,)),
            pltpu.SemaphoreType.DMA((2,)),
            pltpu.SemaphoreType.DMA((2,)),
            pltpu.SemaphoreType.DMA((2,)),
            pltpu.SemaphoreType.DMA((2,)),
            pltpu.SemaphoreType.REGULAR,
            pltpu.SemaphoreType.REGULAR,
        ],
    )
    return pl.pallas_call(
        body,
        out_shape=out_shape,
        grid_spec=grid_spec,
        compiler_params=pltpu.CompilerParams(collective_id=0),
    )(x, Wq, K_ext, V_ext, Wo)


# baseline (device time: 928423 ns/iter reference)
import functools

import jax
import jax.numpy as jnp
from jax import lax
from jax.experimental import pallas as pl
from jax.experimental.pallas import tpu as pltpu

N_DEV = 8
SQ = 1024
SKV_LOC = 1024
HQ_LOC = 8
DH = 128
NG = 32
BAND_KV = 1152
SCALE = 0.08838834764831843

_sig = getattr(pl, "semaphore_signal", None) or pltpu.semaphore_signal
_swait = getattr(pl, "semaphore_wait", None) or pltpu.semaphore_wait

MESH = pl.DeviceIdType.MESH


def kernel(x, Wq, K_ext, V_ext, Wo):
    def body(x_ref, wq_ref, k_ref, v_ref, wo_ref, out_ref,
             q_sc, qg_all, kc_sc, vc_sc, mine_nd, s_nd, comm_nd,
             kb_sc, vb_sc, partial_sc, comm_out,
             qg_send, qg_recv, scat_send, kbvb_recv, loc_sems, fill_sems,
             ring_send, ring_recv, credit_sem, minibar_sem):
        pos = lax.axis_index("i")
        left = lax.rem(pos + N_DEV - 1, N_DEV)
        right = lax.rem(pos + 1, N_DEV)

        barrier_sem = pltpu.get_barrier_semaphore()
        for off in range(1, N_DEV):
            _sig(barrier_sem, inc=1, device_id=(lax.rem(pos + off, N_DEV),),
                 device_id_type=MESH)
        _swait(barrier_sem, N_DEV - 1)

        def scatter_rdmas(src_pos):
            rdmas = []
            if src_pos == 0:
                ksl = lambda j: k_ref.at[0, :, 8 * j:8 * j + 8, :]
                vsl = lambda j: v_ref.at[0, :, 8 * j:8 * j + 8, :]
                kdst = kb_sc.at[0:1024]
                vdst = vb_sc.at[0:1024]
                ksem, vsem = 0, 2
            else:
                ksl = lambda j: k_ref.at[0, 0:128, 8 * j:8 * j + 8, :]
                vsl = lambda j: v_ref.at[0, 0:128, 8 * j:8 * j + 8, :]
                kdst = kb_sc.at[1024:1152]
                vdst = vb_sc.at[1024:1152]
                ksem, vsem = 1, 3
            s = 0
            for j in range(N_DEV):
                if j == src_pos:
                    continue
                rdmas.append(pltpu.make_async_remote_copy(
                    src_ref=ksl(j), dst_ref=kdst,
                    send_sem=scat_send.at[s], recv_sem=kbvb_recv.at[ksem],
                    device_id=(j,), device_id_type=MESH))
                rdmas.append(pltpu.make_async_remote_copy(
                    src_ref=vsl(j), dst_ref=vdst,
                    send_sem=scat_send.at[s + 7], recv_sem=kbvb_recv.at[vsem],
                    device_id=(j,), device_id_type=MESH))
                s += 1
            return rdmas

        def ring_allreduce(mine_ref, s_ref, comm_ref):
            for h in range(N_DEV - 1):
                slot = h % 2
                rdma = pltpu.make_async_remote_copy(
                    src_ref=s_ref, dst_ref=comm_ref.at[slot],
                    send_sem=ring_send.at[slot], recv_sem=ring_recv.at[slot],
                    device_id=(right,), device_id_type=MESH)
                if h >= 2:
                    _swait(credit_sem, 1)
                rdma.start()
                rdma.wait_send()
                rdma.wait_recv()
                s_ref[...] = comm_ref[slot] + mine_ref[...]
                if h < N_DEV - 3:
                    _sig(credit_sem, inc=1, device_id=(left,),
                         device_id_type=MESH)

        def neighbor_barrier():
            _sig(minibar_sem, inc=1, device_id=(left,), device_id_type=MESH)
            _sig(minibar_sem, inc=1, device_id=(right,), device_id_type=MESH)
            _swait(minibar_sem, 2)

        @pl.when(pos == 0)
        def _():
            for r in scatter_rdmas(0):
                r.start()
            pltpu.make_async_copy(k_ref.at[0, :, 0:8, :], kb_sc.at[0:1024],
                                  fill_sems.at[0]).start()
            pltpu.make_async_copy(v_ref.at[0, :, 0:8, :], vb_sc.at[0:1024],
                                  fill_sems.at[1]).start()

        @pl.when(pos == 1)
        def _():
            for r in scatter_rdmas(1):
                r.start()
            pltpu.make_async_copy(k_ref.at[0, 0:128, 8:16, :],
                                  kb_sc.at[1024:1152], fill_sems.at[0]).start()
            pltpu.make_async_copy(v_ref.at[0, 0:128, 8:16, :],
                                  vb_sc.at[1024:1152], fill_sems.at[1]).start()

        q = jnp.dot(x_ref[0], wq_ref[...], preferred_element_type=jnp.float32)
        q_sc[...] = q
        qg_all[pl.ds(pos, 1)] = q[0:32, :][None]

        qg_rdmas = []
        for off in range(1, N_DEV):
            tgt = lax.rem(pos + off, N_DEV)
            r = pltpu.make_async_remote_copy(
                src_ref=qg_all.at[pl.ds(pos, 1)],
                dst_ref=qg_all.at[pl.ds(pos, 1)],
                send_sem=qg_send.at[off], recv_sem=qg_recv.at[off],
                device_id=(tgt,), device_id_type=MESH)
            r.start()
            qg_rdmas.append(r)
        for offb in range(1, N_DEV):
            src = lax.rem(pos + N_DEV - offb, N_DEV)
            pltpu.make_async_remote_copy(
                src_ref=qg_all.at[pl.ds(pos, 1)],
                dst_ref=qg_all.at[pl.ds(src, 1)],
                send_sem=qg_send.at[offb], recv_sem=qg_recv.at[offb],
                device_id=(src,), device_id_type=MESH).wait_recv()

        qga = qg_all[...]
        qgt = qga.reshape(N_DEV, NG, HQ_LOC, DH)
        qgt = qgt.transpose(0, 2, 1, 3).reshape(64, NG, DH)

        num = jnp.zeros((64, NG, DH), jnp.float32)
        den = jnp.zeros((64, NG), jnp.float32)
        n_chunks = 8
        ck = SKV_LOC // n_chunks
        for c in range(n_chunks):
            pltpu.make_async_copy(k_ref.at[0, pl.ds(c * ck, ck), :, :],
                                  kc_sc, loc_sems.at[0]).start()
            pltpu.make_async_copy(v_ref.at[0, pl.ds(c * ck, ck), :, :],
                                  vc_sc, loc_sems.at[1]).start()
            pltpu.make_async_copy(k_ref.at[0, pl.ds(c * ck, ck), :, :],
                                  kc_sc, loc_sems.at[0]).wait()
            pltpu.make_async_copy(v_ref.at[0, pl.ds(c * ck, ck), :, :],
                                  vc_sc, loc_sems.at[1]).wait()
            kc = kc_sc[...].transpose(1, 0, 2)
            vc = vc_sc[...].transpose(1, 0, 2)
            sc = jax.lax.dot_general(
                qgt, kc, (((2,), (2,)), ((0,), (0,))),
                preferred_element_type=jnp.float32) * SCALE
            w = jnp.exp(sc)
            num = num + jax.lax.dot_general(
                w, vc, (((2,), (1,)), ((0,), (0,))),
                preferred_element_type=jnp.float32)
            den = den + jnp.sum(w, axis=2)

        den_pad = jnp.pad(den[:, None, :], ((0, 0), (0, 0), (0, DH - NG)))
        packed = jnp.concatenate([num, den_pad], axis=1)
        mine_nd[...] = packed
        s_nd[...] = packed

        for r in qg_rdmas:
            r.wait_send()

        ring_allreduce(mine_nd, s_nd, comm_nd)

        myslice = s_nd[pl.ds(pos * HQ_LOC, HQ_LOC)]
        g_num = myslice[:, 0:NG, :]
        g_den = myslice[:, NG, 0:NG]

        @pl.when(pos == 0)
        def _():
            for r in scatter_rdmas(0):
                r.wait_send()
            pltpu.make_async_copy(k_ref.at[0, :, 0:8, :], kb_sc.at[0:1024],
                                  fill_sems.at[0]).wait()
            pltpu.make_async_copy(v_ref.at[0, :, 0:8, :], vb_sc.at[0:1024],
                                  fill_sems.at[1]).wait()

        @pl.when(pos == 1)
        def _():
            for r in scatter_rdmas(1):
                r.wait_send()
            pltpu.make_async_copy(k_ref.at[0, 0:128, 8:16, :],
                                  kb_sc.at[1024:1152], fill_sems.at[0]).wait()
            pltpu.make_async_copy(v_ref.at[0, 0:128, 8:16, :],
                                  vb_sc.at[1024:1152], fill_sems.at[1]).wait()

        def recv_band(sem_i, dst, nrows):
            pltpu.make_async_remote_copy(
                src_ref=dst, dst_ref=dst,
                send_sem=scat_send.at[14], recv_sem=kbvb_recv.at[sem_i],
                device_id=(0,), device_id_type=MESH).wait_recv()

        @pl.when(pos != 0)
        def _():
            recv_band(0, kb_sc.at[0:1024], 1024)
            recv_band(2, vb_sc.at[0:1024], 1024)

        @pl.when(pos != 1)
        def _():
            recv_band(1, kb_sc.at[1024:1152], 128)
            recv_band(3, vb_sc.at[1024:1152], 128)

        qi = lax.broadcasted_iota(jnp.int32, (SQ - NG, BAND_KV), 0) + NG
        ki = lax.broadcasted_iota(jnp.int32, (SQ - NG, BAND_KV), 1)
        mask = (jnp.abs(qi - ki) <= 128) | (ki < 32)

        acc = jnp.zeros((SQ, 1024), jnp.float32)
        for hh in range(HQ_LOC):
            qb = q_sc[NG:, hh * DH:(hh + 1) * DH]
            kbh = kb_sc[:, hh, :]
            vbh = vb_sc[:, hh, :]
            sc = jax.lax.dot_general(
                qb, kbh, (((1,), (1,)), ((), ())),
                preferred_element_type=jnp.float32) * SCALE
            sc = jnp.where(mask, sc, -1e9)
            mx = jnp.max(sc, axis=1, keepdims=True)
            w = jnp.exp(sc - mx)
            w = w / jnp.sum(w, axis=1, keepdims=True)
            ctx_band = jax.lax.dot_general(
                w, vbh, (((1,), (0,)), ((), ())),
                preferred_element_type=jnp.float32)
            ctx_g = g_num[hh] / g_den[hh][:, None]
            ctx_h = jnp.concatenate([ctx_g, ctx_band], axis=0)
            acc = acc + jnp.dot(ctx_h, wo_ref[hh * DH:(hh + 1) * DH, :],
                                preferred_element_type=jnp.float32)
        partial_sc[...] = acc

        for r in range(4):
            neighbor_barrier()
            rows = pl.ds(r * 256, 256)
            out_ref[0, rows, :] = partial_sc[rows, :]
            ring_allreduce(partial_sc.at[rows], out_ref.at[0, rows, :],
                           comm_out)

    out_shape = jax.ShapeDtypeStruct((1, SQ, 1024), jnp.float32)
    return pl.pallas_call(
        body,
        out_shape=out_shape,
        in_specs=[
            pl.BlockSpec(memory_space=pltpu.VMEM),
            pl.BlockSpec(memory_space=pltpu.VMEM),
            pl.BlockSpec(memory_space=pl.ANY),
            pl.BlockSpec(memory_space=pl.ANY),
            pl.BlockSpec(memory_space=pltpu.VMEM),
        ],
        out_specs=pl.BlockSpec(memory_space=pltpu.VMEM),
        scratch_shapes=[
            pltpu.VMEM((SQ, 1024), jnp.float32),
            pltpu.VMEM((N_DEV, NG, 1024), jnp.float32),
            pltpu.VMEM((128, 64, DH), jnp.float32),
            pltpu.VMEM((128, 64, DH), jnp.float32),
            pltpu.VMEM((64, NG + 1, DH), jnp.float32),
            pltpu.VMEM((64, NG + 1, DH), jnp.float32),
            pltpu.VMEM((2, 64, NG + 1, DH), jnp.float32),
            pltpu.VMEM((BAND_KV, HQ_LOC, DH), jnp.float32),
            pltpu.VMEM((BAND_KV, HQ_LOC, DH), jnp.float32),
            pltpu.VMEM((SQ, 1024), jnp.float32),
            pltpu.VMEM((2, 256, 1024), jnp.float32),
            pltpu.SemaphoreType.DMA((N_DEV,)),
            pltpu.SemaphoreType.DMA((N_DEV,)),
            pltpu.SemaphoreType.DMA((16,)),
            pltpu.SemaphoreType.DMA((4,)),
            pltpu.SemaphoreType.DMA((2,)),
            pltpu.SemaphoreType.DMA((2,)),
            pltpu.SemaphoreType.DMA((2,)),
            pltpu.SemaphoreType.DMA((2,)),
            pltpu.SemaphoreType.REGULAR,
            pltpu.SemaphoreType.REGULAR,
        ],
        compiler_params=pltpu.CompilerParams(
            collective_id=0, vmem_limit_bytes=60 * 2**20),
    )(x, Wq, K_ext, V_ext, Wo)


# device time: 586449 ns/iter; 1.5831x vs baseline; 1.5831x over previous
import functools

import jax
import jax.numpy as jnp
from jax import lax
from jax.experimental import pallas as pl
from jax.experimental.pallas import tpu as pltpu

N_DEV = 8
SQ = 1024
SKV_LOC = 1024
HQ_LOC = 8
DH = 128
NG = 32
BAND_KV = 1152
SCALE = 0.08838834764831843

_sig = getattr(pl, "semaphore_signal", None) or pltpu.semaphore_signal
_swait = getattr(pl, "semaphore_wait", None) or pltpu.semaphore_wait

MESH = pl.DeviceIdType.MESH


def kernel(x, Wq, K_ext, V_ext, Wo):
    def body(x_ref, wq_ref, k_ref, v_ref, wo_ref, out_ref,
             q_sc, qg_all, kc_sc, vc_sc, nd_all, nd_stage, nd_comm,
             kb_sc, vb_sc, partial_sc, rs_stage, rs_comm,
             qg_send, qg_recv, scat_send, kbvb_recv, loc_sems, fill_sems,
             ring_send, ring_recv, credit_sem, minibar_sem):
        pos = lax.axis_index("i")
        left = lax.rem(pos + N_DEV - 1, N_DEV)
        right = lax.rem(pos + 1, N_DEV)

        barrier_sem = pltpu.get_barrier_semaphore()
        for off in range(1, N_DEV):
            _sig(barrier_sem, inc=1, device_id=(lax.rem(pos + off, N_DEV),),
                 device_id_type=MESH)
        _swait(barrier_sem, N_DEV - 1)

        def scatter_rdmas(src_pos):
            rdmas = []
            if src_pos == 0:
                ksl = lambda j: k_ref.at[0, :, 8 * j:8 * j + 8, :]
                vsl = lambda j: v_ref.at[0, :, 8 * j:8 * j + 8, :]
                kdst = kb_sc.at[0:1024]
                vdst = vb_sc.at[0:1024]
                ksem, vsem = 0, 2
            else:
                ksl = lambda j: k_ref.at[0, 0:128, 8 * j:8 * j + 8, :]
                vsl = lambda j: v_ref.at[0, 0:128, 8 * j:8 * j + 8, :]
                kdst = kb_sc.at[1024:1152]
                vdst = vb_sc.at[1024:1152]
                ksem, vsem = 1, 3
            s = 0
            for j in range(N_DEV):
                if j == src_pos:
                    continue
                rdmas.append(pltpu.make_async_remote_copy(
                    src_ref=ksl(j), dst_ref=kdst,
                    send_sem=scat_send.at[s], recv_sem=kbvb_recv.at[ksem],
                    device_id=(j,), device_id_type=MESH))
                rdmas.append(pltpu.make_async_remote_copy(
                    src_ref=vsl(j), dst_ref=vdst,
                    send_sem=scat_send.at[s + 7], recv_sem=kbvb_recv.at[vsem],
                    device_id=(j,), device_id_type=MESH))
                s += 1
            return rdmas

        def ring_reduce_scatter(part_ref, stage_ref, comm_ref, rows):
            rdmas = []
            for h in range(N_DEV - 1):
                slot = h % 2
                sg = lax.rem(pos + (2 * N_DEV - 1 - h), N_DEV)
                val = part_ref[pl.ds(sg * rows, rows)]
                if h > 0:
                    val = val + comm_ref[(h - 1) % 2]
                if h >= 2:
                    rdmas[h - 2].wait_send()
                stage_ref[slot] = val
                if 1 <= h <= N_DEV - 3:
                    _sig(credit_sem, inc=1, device_id=(left,),
                         device_id_type=MESH)
                rdma = pltpu.make_async_remote_copy(
                    src_ref=stage_ref.at[slot], dst_ref=comm_ref.at[slot],
                    send_sem=ring_send.at[slot], recv_sem=ring_recv.at[slot],
                    device_id=(right,), device_id_type=MESH)
                if h >= 2:
                    _swait(credit_sem, 1)
                rdma.start()
                rdmas.append(rdma)
                rdma.wait_recv()
            for r in rdmas[-2:]:
                r.wait_send()
            return comm_ref[(N_DEV - 2) % 2] + part_ref[
                pl.ds(pos * rows, rows)]

        def ring_allgather_out(rows):
            rdmas = []
            for h in range(N_DEV - 1):
                slot = h % 2
                ss = lax.rem(pos + (N_DEV - h), N_DEV)
                if h >= 1:
                    rr_prev = lax.rem(pos + (2 * N_DEV - h), N_DEV)
                    pltpu.make_async_remote_copy(
                        src_ref=out_ref.at[0, pl.ds(0, rows), :],
                        dst_ref=out_ref.at[0, pl.ds(rr_prev * rows, rows), :],
                        send_sem=ring_send.at[(h - 1) % 2],
                        recv_sem=ring_recv.at[(h - 1) % 2],
                        device_id=(left,), device_id_type=MESH).wait_recv()
                    if h <= N_DEV - 3:
                        _sig(credit_sem, inc=1, device_id=(left,),
                             device_id_type=MESH)
                if h >= 2:
                    rdmas[h - 2].wait_send()
                    _swait(credit_sem, 1)
                rdma = pltpu.make_async_remote_copy(
                    src_ref=out_ref.at[0, pl.ds(ss * rows, rows), :],
                    dst_ref=out_ref.at[0, pl.ds(ss * rows, rows), :],
                    send_sem=ring_send.at[slot], recv_sem=ring_recv.at[slot],
                    device_id=(right,), device_id_type=MESH)
                rdma.start()
                rdmas.append(rdma)
            rr_last = lax.rem(pos + N_DEV + 1, N_DEV)
            pltpu.make_async_remote_copy(
                src_ref=out_ref.at[0, pl.ds(0, rows), :],
                dst_ref=out_ref.at[0, pl.ds(rr_last * rows, rows), :],
                send_sem=ring_send.at[(N_DEV - 2) % 2],
                recv_sem=ring_recv.at[(N_DEV - 2) % 2],
                device_id=(left,), device_id_type=MESH).wait_recv()
            for r in rdmas[-2:]:
                r.wait_send()

        def neighbor_barrier():
            _sig(minibar_sem, inc=1, device_id=(left,), device_id_type=MESH)
            _sig(minibar_sem, inc=1, device_id=(right,), device_id_type=MESH)
            _swait(minibar_sem, 2)

        @pl.when(pos == 0)
        def _():
            for r in scatter_rdmas(0):
                r.start()
            pltpu.make_async_copy(k_ref.at[0, :, 0:8, :], kb_sc.at[0:1024],
                                  fill_sems.at[0]).start()
            pltpu.make_async_copy(v_ref.at[0, :, 0:8, :], vb_sc.at[0:1024],
                                  fill_sems.at[1]).start()

        @pl.when(pos == 1)
        def _():
            for r in scatter_rdmas(1):
                r.start()
            pltpu.make_async_copy(k_ref.at[0, 0:128, 8:16, :],
                                  kb_sc.at[1024:1152], fill_sems.at[0]).start()
            pltpu.make_async_copy(v_ref.at[0, 0:128, 8:16, :],
                                  vb_sc.at[1024:1152], fill_sems.at[1]).start()

        q = jnp.dot(x_ref[0], wq_ref[...], preferred_element_type=jnp.float32)
        q_sc[...] = q
        qg_all[pl.ds(pos, 1)] = q[0:32, :][None]

        qg_rdmas = []
        for off in range(1, N_DEV):
            tgt = lax.rem(pos + off, N_DEV)
            r = pltpu.make_async_remote_copy(
                src_ref=qg_all.at[pl.ds(pos, 1)],
                dst_ref=qg_all.at[pl.ds(pos, 1)],
                send_sem=qg_send.at[off], recv_sem=qg_recv.at[off],
                device_id=(tgt,), device_id_type=MESH)
            r.start()
            qg_rdmas.append(r)
        for offb in range(1, N_DEV):
            src = lax.rem(pos + N_DEV - offb, N_DEV)
            pltpu.make_async_remote_copy(
                src_ref=qg_all.at[pl.ds(pos, 1)],
                dst_ref=qg_all.at[pl.ds(src, 1)],
                send_sem=qg_send.at[offb], recv_sem=qg_recv.at[offb],
                device_id=(src,), device_id_type=MESH).wait_recv()

        qga = qg_all[...]
        qgt = qga.reshape(N_DEV, NG, HQ_LOC, DH)
        qgt = qgt.transpose(0, 2, 1, 3).reshape(64, NG, DH)

        num = jnp.zeros((64, NG, DH), jnp.float32)
        den = jnp.zeros((64, NG), jnp.float32)
        n_chunks = 8
        ck = SKV_LOC // n_chunks
        for c in range(n_chunks):
            pltpu.make_async_copy(k_ref.at[0, pl.ds(c * ck, ck), :, :],
                                  kc_sc, loc_sems.at[0]).start()
            pltpu.make_async_copy(v_ref.at[0, pl.ds(c * ck, ck), :, :],
                                  vc_sc, loc_sems.at[1]).start()
            pltpu.make_async_copy(k_ref.at[0, pl.ds(c * ck, ck), :, :],
                                  kc_sc, loc_sems.at[0]).wait()
            pltpu.make_async_copy(v_ref.at[0, pl.ds(c * ck, ck), :, :],
                                  vc_sc, loc_sems.at[1]).wait()
            kc = kc_sc[...].transpose(1, 0, 2)
            vc = vc_sc[...].transpose(1, 0, 2)
            sc = jax.lax.dot_general(
                qgt, kc, (((2,), (2,)), ((0,), (0,))),
                preferred_element_type=jnp.float32) * SCALE
            w = jnp.exp(sc)
            num = num + jax.lax.dot_general(
                w, vc, (((2,), (1,)), ((0,), (0,))),
                preferred_element_type=jnp.float32)
            den = den + jnp.sum(w, axis=2)

        den_pad = jnp.pad(den[:, None, :], ((0, 0), (0, 0), (0, DH - NG)))
        packed = jnp.concatenate([num, den_pad], axis=1)
        nd_all[...] = packed

        for r in qg_rdmas:
            r.wait_send()

        my_nd = ring_reduce_scatter(nd_all, nd_stage, nd_comm, HQ_LOC)
        g_num = my_nd[:, 0:NG, :]
        g_den = my_nd[:, NG, 0:NG]

        @pl.when(pos == 0)
        def _():
            for r in scatter_rdmas(0):
                r.wait_send()
            pltpu.make_async_copy(k_ref.at[0, :, 0:8, :], kb_sc.at[0:1024],
                                  fill_sems.at[0]).wait()
            pltpu.make_async_copy(v_ref.at[0, :, 0:8, :], vb_sc.at[0:1024],
                                  fill_sems.at[1]).wait()

        @pl.when(pos == 1)
        def _():
            for r in scatter_rdmas(1):
                r.wait_send()
            pltpu.make_async_copy(k_ref.at[0, 0:128, 8:16, :],
                                  kb_sc.at[1024:1152], fill_sems.at[0]).wait()
            pltpu.make_async_copy(v_ref.at[0, 0:128, 8:16, :],
                                  vb_sc.at[1024:1152], fill_sems.at[1]).wait()

        def recv_band(sem_i, dst, nrows):
            pltpu.make_async_remote_copy(
                src_ref=dst, dst_ref=dst,
                send_sem=scat_send.at[14], recv_sem=kbvb_recv.at[sem_i],
                device_id=(0,), device_id_type=MESH).wait_recv()

        @pl.when(pos != 0)
        def _():
            recv_band(0, kb_sc.at[0:1024], 1024)
            recv_band(2, vb_sc.at[0:1024], 1024)

        @pl.when(pos != 1)
        def _():
            recv_band(1, kb_sc.at[1024:1152], 128)
            recv_band(3, vb_sc.at[1024:1152], 128)

        qi = lax.broadcasted_iota(jnp.int32, (SQ - NG, BAND_KV), 0) + NG
        ki = lax.broadcasted_iota(jnp.int32, (SQ - NG, BAND_KV), 1)
        mask = (jnp.abs(qi - ki) <= 128) | (ki < 32)

        acc = jnp.zeros((SQ, 1024), jnp.float32)
        for hh in range(HQ_LOC):
            qb = q_sc[NG:, hh * DH:(hh + 1) * DH]
            kbh = kb_sc[:, hh, :]
            vbh = vb_sc[:, hh, :]
            sc = jax.lax.dot_general(
                qb, kbh, (((1,), (1,)), ((), ())),
                preferred_element_type=jnp.float32) * SCALE
            sc = jnp.where(mask, sc, -1e9)
            mx = jnp.max(sc, axis=1, keepdims=True)
            w = jnp.exp(sc - mx)
            w = w / jnp.sum(w, axis=1, keepdims=True)
            ctx_band = jax.lax.dot_general(
                w, vbh, (((1,), (0,)), ((), ())),
                preferred_element_type=jnp.float32)
            ctx_g = g_num[hh] / g_den[hh][:, None]
            ctx_h = jnp.concatenate([ctx_g, ctx_band], axis=0)
            acc = acc + jnp.dot(ctx_h, wo_ref[hh * DH:(hh + 1) * DH, :],
                                preferred_element_type=jnp.float32)
        partial_sc[...] = acc

        neighbor_barrier()
        fin = ring_reduce_scatter(partial_sc, rs_stage, rs_comm, 128)
        out_ref[0, pl.ds(pos * 128, 128), :] = fin
        neighbor_barrier()
        ring_allgather_out(128)

    out_shape = jax.ShapeDtypeStruct((1, SQ, 1024), jnp.float32)
    return pl.pallas_call(
        body,
        out_shape=out_shape,
        in_specs=[
            pl.BlockSpec(memory_space=pltpu.VMEM),
            pl.BlockSpec(memory_space=pltpu.VMEM),
            pl.BlockSpec(memory_space=pl.ANY),
            pl.BlockSpec(memory_space=pl.ANY),
            pl.BlockSpec(memory_space=pltpu.VMEM),
        ],
        out_specs=pl.BlockSpec(memory_space=pltpu.VMEM),
        scratch_shapes=[
            pltpu.VMEM((SQ, 1024), jnp.float32),
            pltpu.VMEM((N_DEV, NG, 1024), jnp.float32),
            pltpu.VMEM((128, 64, DH), jnp.float32),
            pltpu.VMEM((128, 64, DH), jnp.float32),
            pltpu.VMEM((64, NG + 1, DH), jnp.float32),
            pltpu.VMEM((2, HQ_LOC, NG + 1, DH), jnp.float32),
            pltpu.VMEM((2, HQ_LOC, NG + 1, DH), jnp.float32),
            pltpu.VMEM((BAND_KV, HQ_LOC, DH), jnp.float32),
            pltpu.VMEM((BAND_KV, HQ_LOC, DH), jnp.float32),
            pltpu.VMEM((SQ, 1024), jnp.float32),
            pltpu.VMEM((2, 128, 1024), jnp.float32),
            pltpu.VMEM((2, 128, 1024), jnp.float32),
            pltpu.SemaphoreType.DMA((N_DEV,)),
            pltpu.SemaphoreType.DMA((N_DEV,)),
            pltpu.SemaphoreType.DMA((16,)),
            pltpu.SemaphoreType.DMA((4,)),
            pltpu.SemaphoreType.DMA((2,)),
            pltpu.SemaphoreType.DMA((2,)),
            pltpu.SemaphoreType.DMA((2,)),
            pltpu.SemaphoreType.DMA((2,)),
            pltpu.SemaphoreType.REGULAR,
            pltpu.SemaphoreType.REGULAR,
        ],
        compiler_params=pltpu.CompilerParams(
            collective_id=0, vmem_limit_bytes=60 * 2**20),
    )(x, Wq, K_ext, V_ext, Wo)


# device time: 531152 ns/iter; 1.7479x vs baseline; 1.1041x over previous
import functools

import jax
import jax.numpy as jnp
from jax import lax
from jax.experimental import pallas as pl
from jax.experimental.pallas import tpu as pltpu

N_DEV = 8
SQ = 1024
SKV_LOC = 1024
HQ_LOC = 8
DH = 128
NG = 32
BAND_KV = 1152
SCALE = 0.08838834764831843

_sig = getattr(pl, "semaphore_signal", None) or pltpu.semaphore_signal
_swait = getattr(pl, "semaphore_wait", None) or pltpu.semaphore_wait

MESH = pl.DeviceIdType.MESH


def kernel(x, Wq, K_ext, V_ext, Wo):
    def body(x_ref, wq_ref, k_ref, v_ref, wo_ref, out_ref,
             q_sc, qg_all, kc_sc, vc_sc, nd_all, nd_stage, nd_comm,
             kb_sc, vb_sc, partial_sc, rs_stage, rs_comm, ag_stage, ag_comm,
             qg_send, qg_recv, scat_send, kbvb_recv, loc_sems, fill_sems,
             ring_send, ring_recv, credit_sem, minibar_sem):
        pos = lax.axis_index("i")
        left = lax.rem(pos + N_DEV - 1, N_DEV)
        right = lax.rem(pos + 1, N_DEV)

        barrier_sem = pltpu.get_barrier_semaphore()
        for off in range(1, N_DEV):
            _sig(barrier_sem, inc=1, device_id=(lax.rem(pos + off, N_DEV),),
                 device_id_type=MESH)
        _swait(barrier_sem, N_DEV - 1)

        def scatter_rdmas(src_pos):
            rdmas = []
            if src_pos == 0:
                ksl = lambda j: k_ref.at[0, :, 8 * j:8 * j + 8, :]
                vsl = lambda j: v_ref.at[0, :, 8 * j:8 * j + 8, :]
                kdst = kb_sc.at[0:1024]
                vdst = vb_sc.at[0:1024]
                ksem, vsem = 0, 2
            else:
                ksl = lambda j: k_ref.at[0, 0:128, 8 * j:8 * j + 8, :]
                vsl = lambda j: v_ref.at[0, 0:128, 8 * j:8 * j + 8, :]
                kdst = kb_sc.at[1024:1152]
                vdst = vb_sc.at[1024:1152]
                ksem, vsem = 1, 3
            s = 0
            for j in range(N_DEV):
                if j == src_pos:
                    continue
                rdmas.append(pltpu.make_async_remote_copy(
                    src_ref=ksl(j), dst_ref=kdst,
                    send_sem=scat_send.at[s], recv_sem=kbvb_recv.at[ksem],
                    device_id=(j,), device_id_type=MESH))
                rdmas.append(pltpu.make_async_remote_copy(
                    src_ref=vsl(j), dst_ref=vdst,
                    send_sem=scat_send.at[s + 7], recv_sem=kbvb_recv.at[vsem],
                    device_id=(j,), device_id_type=MESH))
                s += 1
            return rdmas

        def ring_reduce_scatter(part_ref, stage_ref, comm_ref, rows):
            rdmas = []
            for h in range(N_DEV - 1):
                slot = h % 2
                sg = lax.rem(pos + (2 * N_DEV - 1 - h), N_DEV)
                val = part_ref[pl.ds(sg * rows, rows)]
                if h > 0:
                    val = val + comm_ref[(h - 1) % 2].astype(jnp.float32)
                if h >= 2:
                    rdmas[h - 2].wait_send()
                stage_ref[slot] = val.astype(jnp.bfloat16)
                if 1 <= h <= N_DEV - 3:
                    _sig(credit_sem, inc=1, device_id=(left,),
                         device_id_type=MESH)
                rdma = pltpu.make_async_remote_copy(
                    src_ref=stage_ref.at[slot], dst_ref=comm_ref.at[slot],
                    send_sem=ring_send.at[slot], recv_sem=ring_recv.at[slot],
                    device_id=(right,), device_id_type=MESH)
                if h >= 2:
                    _swait(credit_sem, 1)
                rdma.start()
                rdmas.append(rdma)
                rdma.wait_recv()
            for r in rdmas[-2:]:
                r.wait_send()
            return comm_ref[(N_DEV - 2) % 2].astype(jnp.float32) + part_ref[
                pl.ds(pos * rows, rows)]

        def ring_allgather_out(rows):
            rdmas = []
            for h in range(N_DEV - 1):
                slot = h % 2
                src = ag_stage if h == 0 else ag_comm.at[(h - 1) % 2]
                rdma = pltpu.make_async_remote_copy(
                    src_ref=src, dst_ref=ag_comm.at[slot],
                    send_sem=ring_send.at[slot], recv_sem=ring_recv.at[slot],
                    device_id=(right,), device_id_type=MESH)
                if h >= 2:
                    _swait(credit_sem, 1)
                rdma.start()
                rdmas.append(rdma)
                if h >= 1:
                    rdma.wait_send()
                    if h <= N_DEV - 3:
                        _sig(credit_sem, inc=1, device_id=(left,),
                             device_id_type=MESH)
                rdma.wait_recv()
                rr = lax.rem(pos + (2 * N_DEV - 1 - h), N_DEV)
                out_ref[0, pl.ds(rr * rows, rows), :] = (
                    ag_comm[slot].astype(jnp.float32))
            rdmas[0].wait_send()

        def neighbor_barrier():
            _sig(minibar_sem, inc=1, device_id=(left,), device_id_type=MESH)
            _sig(minibar_sem, inc=1, device_id=(right,), device_id_type=MESH)
            _swait(minibar_sem, 2)

        @pl.when(pos == 0)
        def _():
            for r in scatter_rdmas(0):
                r.start()
            pltpu.make_async_copy(k_ref.at[0, :, 0:8, :], kb_sc.at[0:1024],
                                  fill_sems.at[0]).start()
            pltpu.make_async_copy(v_ref.at[0, :, 0:8, :], vb_sc.at[0:1024],
                                  fill_sems.at[1]).start()

        @pl.when(pos == 1)
        def _():
            for r in scatter_rdmas(1):
                r.start()
            pltpu.make_async_copy(k_ref.at[0, 0:128, 8:16, :],
                                  kb_sc.at[1024:1152], fill_sems.at[0]).start()
            pltpu.make_async_copy(v_ref.at[0, 0:128, 8:16, :],
                                  vb_sc.at[1024:1152], fill_sems.at[1]).start()

        q = jnp.dot(x_ref[0], wq_ref[...], preferred_element_type=jnp.float32)
        q_sc[...] = q
        qg_all[pl.ds(pos, 1)] = q[0:32, :][None]

        qg_rdmas = []
        for off in range(1, N_DEV):
            tgt = lax.rem(pos + off, N_DEV)
            r = pltpu.make_async_remote_copy(
                src_ref=qg_all.at[pl.ds(pos, 1)],
                dst_ref=qg_all.at[pl.ds(pos, 1)],
                send_sem=qg_send.at[off], recv_sem=qg_recv.at[off],
                device_id=(tgt,), device_id_type=MESH)
            r.start()
            qg_rdmas.append(r)
        for offb in range(1, N_DEV):
            src = lax.rem(pos + N_DEV - offb, N_DEV)
            pltpu.make_async_remote_copy(
                src_ref=qg_all.at[pl.ds(pos, 1)],
                dst_ref=qg_all.at[pl.ds(src, 1)],
                send_sem=qg_send.at[offb], recv_sem=qg_recv.at[offb],
                device_id=(src,), device_id_type=MESH).wait_recv()

        qga = qg_all[...]
        qgt = qga.reshape(N_DEV, NG, HQ_LOC, DH)
        qgt = qgt.transpose(0, 2, 1, 3).reshape(64, NG, DH)

        num = jnp.zeros((64, NG, DH), jnp.float32)
        den = jnp.zeros((64, NG), jnp.float32)
        n_chunks = 16
        ck = SKV_LOC // n_chunks

        def chunk_copies(c):
            b = c % 2
            return (
                pltpu.make_async_copy(k_ref.at[0, pl.ds(c * ck, ck), :, :],
                                      kc_sc.at[b], loc_sems.at[b]),
                pltpu.make_async_copy(v_ref.at[0, pl.ds(c * ck, ck), :, :],
                                      vc_sc.at[b], loc_sems.at[2 + b]),
            )

        for cp in chunk_copies(0):
            cp.start()
        for c in range(n_chunks):
            if c + 1 < n_chunks:
                for cp in chunk_copies(c + 1):
                    cp.start()
            for cp in chunk_copies(c):
                cp.wait()
            kc = kc_sc[c % 2].transpose(1, 0, 2)
            vc = vc_sc[c % 2].transpose(1, 0, 2)
            sc = jax.lax.dot_general(
                qgt, kc, (((2,), (2,)), ((0,), (0,))),
                preferred_element_type=jnp.float32) * SCALE
            w = jnp.exp(sc)
            num = num + jax.lax.dot_general(
                w, vc, (((2,), (1,)), ((0,), (0,))),
                preferred_element_type=jnp.float32)
            den = den + jnp.sum(w, axis=2)

        den_pad = jnp.pad(den[:, None, :], ((0, 0), (0, 0), (0, DH - NG)))
        packed = jnp.concatenate([num, den_pad], axis=1)
        nd_all[...] = packed

        for r in qg_rdmas:
            r.wait_send()

        my_nd = ring_reduce_scatter(nd_all, nd_stage, nd_comm, HQ_LOC)
        g_num = my_nd[:, 0:NG, :]
        g_den = my_nd[:, NG, 0:NG]

        @pl.when(pos == 0)
        def _():
            for r in scatter_rdmas(0):
                r.wait_send()
            pltpu.make_async_copy(k_ref.at[0, :, 0:8, :], kb_sc.at[0:1024],
                                  fill_sems.at[0]).wait()
            pltpu.make_async_copy(v_ref.at[0, :, 0:8, :], vb_sc.at[0:1024],
                                  fill_sems.at[1]).wait()

        @pl.when(pos == 1)
        def _():
            for r in scatter_rdmas(1):
                r.wait_send()
            pltpu.make_async_copy(k_ref.at[0, 0:128, 8:16, :],
                                  kb_sc.at[1024:1152], fill_sems.at[0]).wait()
            pltpu.make_async_copy(v_ref.at[0, 0:128, 8:16, :],
                                  vb_sc.at[1024:1152], fill_sems.at[1]).wait()

        def recv_band(sem_i, dst, nrows):
            pltpu.make_async_remote_copy(
                src_ref=dst, dst_ref=dst,
                send_sem=scat_send.at[14], recv_sem=kbvb_recv.at[sem_i],
                device_id=(0,), device_id_type=MESH).wait_recv()

        @pl.when(pos != 0)
        def _():
            recv_band(0, kb_sc.at[0:1024], 1024)
            recv_band(2, vb_sc.at[0:1024], 1024)

        @pl.when(pos != 1)
        def _():
            recv_band(1, kb_sc.at[1024:1152], 128)
            recv_band(3, vb_sc.at[1024:1152], 128)

        qi = lax.broadcasted_iota(jnp.int32, (SQ - NG, BAND_KV), 0) + NG
        ki = lax.broadcasted_iota(jnp.int32, (SQ - NG, BAND_KV), 1)
        mask = (jnp.abs(qi - ki) <= 128) | (ki < 32)

        acc = jnp.zeros((SQ, 1024), jnp.float32)
        for hh in range(HQ_LOC):
            qb = q_sc[NG:, hh * DH:(hh + 1) * DH]
            kbh = kb_sc[:, hh, :]
            vbh = vb_sc[:, hh, :]
            sc = jax.lax.dot_general(
                qb, kbh, (((1,), (1,)), ((), ())),
                preferred_element_type=jnp.float32) * SCALE
            sc = jnp.where(mask, sc, -1e9)
            mx = jnp.max(sc, axis=1, keepdims=True)
            w = jnp.exp(sc - mx)
            w = w / jnp.sum(w, axis=1, keepdims=True)
            ctx_band = jax.lax.dot_general(
                w, vbh, (((1,), (0,)), ((), ())),
                preferred_element_type=jnp.float32)
            ctx_g = g_num[hh] / g_den[hh][:, None]
            ctx_h = jnp.concatenate([ctx_g, ctx_band], axis=0)
            acc = acc + jnp.dot(ctx_h, wo_ref[hh * DH:(hh + 1) * DH, :],
                                preferred_element_type=jnp.float32)
        partial_sc[...] = acc

        neighbor_barrier()
        fin = ring_reduce_scatter(partial_sc, rs_stage, rs_comm, 128)
        out_ref[0, pl.ds(pos * 128, 128), :] = fin
        ag_stage[...] = fin.astype(jnp.bfloat16)
        neighbor_barrier()
        ring_allgather_out(128)

    out_shape = jax.ShapeDtypeStruct((1, SQ, 1024), jnp.float32)
    return pl.pallas_call(
        body,
        out_shape=out_shape,
        in_specs=[
            pl.BlockSpec(memory_space=pltpu.VMEM),
            pl.BlockSpec(memory_space=pltpu.VMEM),
            pl.BlockSpec(memory_space=pl.ANY),
            pl.BlockSpec(memory_space=pl.ANY),
            pl.BlockSpec(memory_space=pltpu.VMEM),
        ],
        out_specs=pl.BlockSpec(memory_space=pltpu.VMEM),
        scratch_shapes=[
            pltpu.VMEM((SQ, 1024), jnp.float32),
            pltpu.VMEM((N_DEV, NG, 1024), jnp.float32),
            pltpu.VMEM((2, 64, 64, DH), jnp.float32),
            pltpu.VMEM((2, 64, 64, DH), jnp.float32),
            pltpu.VMEM((64, NG + 1, DH), jnp.float32),
            pltpu.VMEM((2, HQ_LOC, NG + 1, DH), jnp.bfloat16),
            pltpu.VMEM((2, HQ_LOC, NG + 1, DH), jnp.bfloat16),
            pltpu.VMEM((BAND_KV, HQ_LOC, DH), jnp.float32),
            pltpu.VMEM((BAND_KV, HQ_LOC, DH), jnp.float32),
            pltpu.VMEM((SQ, 1024), jnp.float32),
            pltpu.VMEM((2, 128, 1024), jnp.bfloat16),
            pltpu.VMEM((2, 128, 1024), jnp.bfloat16),
            pltpu.VMEM((128, 1024), jnp.bfloat16),
            pltpu.VMEM((2, 128, 1024), jnp.bfloat16),
            pltpu.SemaphoreType.DMA((N_DEV,)),
            pltpu.SemaphoreType.DMA((N_DEV,)),
            pltpu.SemaphoreType.DMA((16,)),
            pltpu.SemaphoreType.DMA((4,)),
            pltpu.SemaphoreType.DMA((4,)),
            pltpu.SemaphoreType.DMA((2,)),
            pltpu.SemaphoreType.DMA((2,)),
            pltpu.SemaphoreType.DMA((2,)),
            pltpu.SemaphoreType.REGULAR,
            pltpu.SemaphoreType.REGULAR,
        ],
        compiler_params=pltpu.CompilerParams(
            collective_id=0, vmem_limit_bytes=60 * 2**20),
    )(x, Wq, K_ext, V_ext, Wo)


# device time: 528168 ns/iter; 1.7578x vs baseline; 1.0056x over previous
import functools

import jax
import jax.numpy as jnp
from jax import lax
from jax.experimental import pallas as pl
from jax.experimental.pallas import tpu as pltpu

N_DEV = 8
SQ = 1024
SKV_LOC = 1024
HQ_LOC = 8
DH = 128
NG = 32
BAND_KV = 1152
SCALE = 0.08838834764831843

_sig = getattr(pl, "semaphore_signal", None) or pltpu.semaphore_signal
_swait = getattr(pl, "semaphore_wait", None) or pltpu.semaphore_wait

MESH = pl.DeviceIdType.MESH


def kernel(x, Wq, K_ext, V_ext, Wo):
    def body(x_ref, wq_ref, k_ref, v_ref, wo_ref, out_ref,
             q_sc, qg_all, kc_sc, vc_sc, nd_all, nd_stage, nd_comm,
             kb_sc, vb_sc, partial_sc, rs_stage, rs_comm, ag_stage, ag_comm,
             qg_send, qg_recv, scat_send, kbvb_recv, loc_sems, fill_sems,
             ring_send, ring_recv, credit_sem, minibar_sem):
        pos = lax.axis_index("i")
        left = lax.rem(pos + N_DEV - 1, N_DEV)
        right = lax.rem(pos + 1, N_DEV)

        barrier_sem = pltpu.get_barrier_semaphore()
        for off in range(1, N_DEV):
            _sig(barrier_sem, inc=1, device_id=(lax.rem(pos + off, N_DEV),),
                 device_id_type=MESH)
        _swait(barrier_sem, N_DEV - 1)

        def scatter_rdmas(src_pos):
            rdmas = []
            if src_pos == 0:
                ksl = lambda j: k_ref.at[0, :, 8 * j:8 * j + 8, :]
                vsl = lambda j: v_ref.at[0, :, 8 * j:8 * j + 8, :]
                kdst = kb_sc.at[0:1024]
                vdst = vb_sc.at[0:1024]
                ksem, vsem = 0, 2
            else:
                ksl = lambda j: k_ref.at[0, 0:128, 8 * j:8 * j + 8, :]
                vsl = lambda j: v_ref.at[0, 0:128, 8 * j:8 * j + 8, :]
                kdst = kb_sc.at[1024:1152]
                vdst = vb_sc.at[1024:1152]
                ksem, vsem = 1, 3
            s = 0
            for j in range(N_DEV):
                if j == src_pos:
                    continue
                rdmas.append(pltpu.make_async_remote_copy(
                    src_ref=ksl(j), dst_ref=kdst,
                    send_sem=scat_send.at[s], recv_sem=kbvb_recv.at[ksem],
                    device_id=(j,), device_id_type=MESH))
                rdmas.append(pltpu.make_async_remote_copy(
                    src_ref=vsl(j), dst_ref=vdst,
                    send_sem=scat_send.at[s + 7], recv_sem=kbvb_recv.at[vsem],
                    device_id=(j,), device_id_type=MESH))
                s += 1
            return rdmas

        def ring_reduce_scatter(part_ref, stage_ref, comm_ref, rows):
            rdmas = []
            for h in range(N_DEV - 1):
                slot = h % 2
                sg = lax.rem(pos + (2 * N_DEV - 1 - h), N_DEV)
                val = part_ref[pl.ds(sg * rows, rows)]
                if h > 0:
                    val = val + comm_ref[(h - 1) % 2].astype(jnp.float32)
                if h >= 2:
                    rdmas[h - 2].wait_send()
                stage_ref[slot] = val.astype(jnp.bfloat16)
                if 1 <= h <= N_DEV - 3:
                    _sig(credit_sem, inc=1, device_id=(left,),
                         device_id_type=MESH)
                rdma = pltpu.make_async_remote_copy(
                    src_ref=stage_ref.at[slot], dst_ref=comm_ref.at[slot],
                    send_sem=ring_send.at[slot], recv_sem=ring_recv.at[slot],
                    device_id=(right,), device_id_type=MESH)
                if h >= 2:
                    _swait(credit_sem, 1)
                rdma.start()
                rdmas.append(rdma)
                rdma.wait_recv()
            for r in rdmas[-2:]:
                r.wait_send()
            return comm_ref[(N_DEV - 2) % 2].astype(jnp.float32) + part_ref[
                pl.ds(pos * rows, rows)]

        def ring_allgather_out(rows):
            rdmas = []
            for h in range(N_DEV - 1):
                slot = h % 2
                src = ag_stage if h == 0 else ag_comm.at[(h - 1) % 2]
                rdma = pltpu.make_async_remote_copy(
                    src_ref=src, dst_ref=ag_comm.at[slot],
                    send_sem=ring_send.at[slot], recv_sem=ring_recv.at[slot],
                    device_id=(right,), device_id_type=MESH)
                if h >= 2:
                    _swait(credit_sem, 1)
                rdma.start()
                rdmas.append(rdma)
                if h >= 1:
                    rdma.wait_send()
                    if h <= N_DEV - 3:
                        _sig(credit_sem, inc=1, device_id=(left,),
                             device_id_type=MESH)
                rdma.wait_recv()
                rr = lax.rem(pos + (2 * N_DEV - 1 - h), N_DEV)
                out_ref[0, pl.ds(rr * rows, rows), :] = (
                    ag_comm[slot].astype(jnp.float32))
            rdmas[0].wait_send()

        def neighbor_barrier():
            _sig(minibar_sem, inc=1, device_id=(left,), device_id_type=MESH)
            _sig(minibar_sem, inc=1, device_id=(right,), device_id_type=MESH)
            _swait(minibar_sem, 2)

        @pl.when(pos == 0)
        def _():
            for r in scatter_rdmas(0):
                r.start()
            pltpu.make_async_copy(k_ref.at[0, :, 0:8, :], kb_sc.at[0:1024],
                                  fill_sems.at[0]).start()
            pltpu.make_async_copy(v_ref.at[0, :, 0:8, :], vb_sc.at[0:1024],
                                  fill_sems.at[1]).start()

        @pl.when(pos == 1)
        def _():
            for r in scatter_rdmas(1):
                r.start()
            pltpu.make_async_copy(k_ref.at[0, 0:128, 8:16, :],
                                  kb_sc.at[1024:1152], fill_sems.at[0]).start()
            pltpu.make_async_copy(v_ref.at[0, 0:128, 8:16, :],
                                  vb_sc.at[1024:1152], fill_sems.at[1]).start()

        q = jnp.dot(x_ref[0], wq_ref[...], preferred_element_type=jnp.float32)
        q_sc[...] = q
        qg_all[pl.ds(pos, 1)] = q[0:32, :][None]

        qg_rdmas = []
        for off in range(1, N_DEV):
            tgt = lax.rem(pos + off, N_DEV)
            r = pltpu.make_async_remote_copy(
                src_ref=qg_all.at[pl.ds(pos, 1)],
                dst_ref=qg_all.at[pl.ds(pos, 1)],
                send_sem=qg_send.at[off], recv_sem=qg_recv.at[off],
                device_id=(tgt,), device_id_type=MESH)
            r.start()
            qg_rdmas.append(r)
        for offb in range(1, N_DEV):
            src = lax.rem(pos + N_DEV - offb, N_DEV)
            pltpu.make_async_remote_copy(
                src_ref=qg_all.at[pl.ds(pos, 1)],
                dst_ref=qg_all.at[pl.ds(src, 1)],
                send_sem=qg_send.at[offb], recv_sem=qg_recv.at[offb],
                device_id=(src,), device_id_type=MESH).wait_recv()

        qga = qg_all[...]
        qgt = qga.reshape(N_DEV, NG, HQ_LOC, DH)
        qgt = qgt.transpose(0, 2, 1, 3).reshape(64, NG, DH)

        num = jnp.zeros((64, NG, DH), jnp.float32)
        den = jnp.zeros((64, NG), jnp.float32)
        n_chunks = 16
        ck = SKV_LOC // n_chunks

        def chunk_copies(c):
            b = c % 2
            return (
                pltpu.make_async_copy(k_ref.at[0, pl.ds(c * ck, ck), :, :],
                                      kc_sc.at[b], loc_sems.at[b]),
                pltpu.make_async_copy(v_ref.at[0, pl.ds(c * ck, ck), :, :],
                                      vc_sc.at[b], loc_sems.at[2 + b]),
            )

        for cp in chunk_copies(0):
            cp.start()
        for c in range(n_chunks):
            if c + 1 < n_chunks:
                for cp in chunk_copies(c + 1):
                    cp.start()
            for cp in chunk_copies(c):
                cp.wait()
            kc = kc_sc[c % 2].astype(jnp.bfloat16).transpose(1, 0, 2)
            vc = vc_sc[c % 2].astype(jnp.bfloat16).transpose(1, 0, 2)
            sc = jax.lax.dot_general(
                qgt.astype(jnp.bfloat16), kc, (((2,), (2,)), ((0,), (0,))),
                preferred_element_type=jnp.float32) * SCALE
            w = jnp.exp(sc)
            num = num + jax.lax.dot_general(
                w.astype(jnp.bfloat16), vc, (((2,), (1,)), ((0,), (0,))),
                preferred_element_type=jnp.float32)
            den = den + jnp.sum(w, axis=2)

        den_pad = jnp.pad(den[:, None, :], ((0, 0), (0, 0), (0, DH - NG)))
        packed = jnp.concatenate([num, den_pad], axis=1)
        nd_all[...] = packed

        for r in qg_rdmas:
            r.wait_send()

        my_nd = ring_reduce_scatter(nd_all, nd_stage, nd_comm, HQ_LOC)
        g_num = my_nd[:, 0:NG, :]
        g_den = my_nd[:, NG, 0:NG]

        @pl.when(pos == 0)
        def _():
            for r in scatter_rdmas(0):
                r.wait_send()
            pltpu.make_async_copy(k_ref.at[0, :, 0:8, :], kb_sc.at[0:1024],
                                  fill_sems.at[0]).wait()
            pltpu.make_async_copy(v_ref.at[0, :, 0:8, :], vb_sc.at[0:1024],
                                  fill_sems.at[1]).wait()

        @pl.when(pos == 1)
        def _():
            for r in scatter_rdmas(1):
                r.wait_send()
            pltpu.make_async_copy(k_ref.at[0, 0:128, 8:16, :],
                                  kb_sc.at[1024:1152], fill_sems.at[0]).wait()
            pltpu.make_async_copy(v_ref.at[0, 0:128, 8:16, :],
                                  vb_sc.at[1024:1152], fill_sems.at[1]).wait()

        def recv_band(sem_i, dst, nrows):
            pltpu.make_async_remote_copy(
                src_ref=dst, dst_ref=dst,
                send_sem=scat_send.at[14], recv_sem=kbvb_recv.at[sem_i],
                device_id=(0,), device_id_type=MESH).wait_recv()

        @pl.when(pos != 0)
        def _():
            recv_band(0, kb_sc.at[0:1024], 1024)
            recv_band(2, vb_sc.at[0:1024], 1024)

        @pl.when(pos != 1)
        def _():
            recv_band(1, kb_sc.at[1024:1152], 128)
            recv_band(3, vb_sc.at[1024:1152], 128)

        qi = lax.broadcasted_iota(jnp.int32, (SQ - NG, BAND_KV), 0) + NG
        ki = lax.broadcasted_iota(jnp.int32, (SQ - NG, BAND_KV), 1)
        mask = (jnp.abs(qi - ki) <= 128) | (ki < 32)

        acc = jnp.zeros((SQ, 1024), jnp.float32)
        for hh in range(HQ_LOC):
            qb = q_sc[NG:, hh * DH:(hh + 1) * DH]
            kbh = kb_sc[:, hh, :]
            vbh = vb_sc[:, hh, :]
            sc = jax.lax.dot_general(
                qb, kbh, (((1,), (1,)), ((), ())),
                preferred_element_type=jnp.float32) * SCALE
            sc = jnp.where(mask, sc, -1e9)
            mx = jnp.max(sc, axis=1, keepdims=True)
            w = jnp.exp(sc - mx)
            w = w / jnp.sum(w, axis=1, keepdims=True)
            ctx_band = jax.lax.dot_general(
                w, vbh, (((1,), (0,)), ((), ())),
                preferred_element_type=jnp.float32)
            ctx_g = g_num[hh] / g_den[hh][:, None]
            ctx_h = jnp.concatenate([ctx_g, ctx_band], axis=0)
            acc = acc + jnp.dot(ctx_h, wo_ref[hh * DH:(hh + 1) * DH, :],
                                preferred_element_type=jnp.float32)
        partial_sc[...] = acc

        neighbor_barrier()
        fin = ring_reduce_scatter(partial_sc, rs_stage, rs_comm, 128)
        out_ref[0, pl.ds(pos * 128, 128), :] = fin
        ag_stage[...] = fin.astype(jnp.bfloat16)
        neighbor_barrier()
        ring_allgather_out(128)

    out_shape = jax.ShapeDtypeStruct((1, SQ, 1024), jnp.float32)
    return pl.pallas_call(
        body,
        out_shape=out_shape,
        in_specs=[
            pl.BlockSpec(memory_space=pltpu.VMEM),
            pl.BlockSpec(memory_space=pltpu.VMEM),
            pl.BlockSpec(memory_space=pl.ANY),
            pl.BlockSpec(memory_space=pl.ANY),
            pl.BlockSpec(memory_space=pltpu.VMEM),
        ],
        out_specs=pl.BlockSpec(memory_space=pltpu.VMEM),
        scratch_shapes=[
            pltpu.VMEM((SQ, 1024), jnp.float32),
            pltpu.VMEM((N_DEV, NG, 1024), jnp.float32),
            pltpu.VMEM((2, 64, 64, DH), jnp.float32),
            pltpu.VMEM((2, 64, 64, DH), jnp.float32),
            pltpu.VMEM((64, NG + 1, DH), jnp.float32),
            pltpu.VMEM((2, HQ_LOC, NG + 1, DH), jnp.bfloat16),
            pltpu.VMEM((2, HQ_LOC, NG + 1, DH), jnp.bfloat16),
            pltpu.VMEM((BAND_KV, HQ_LOC, DH), jnp.float32),
            pltpu.VMEM((BAND_KV, HQ_LOC, DH), jnp.float32),
            pltpu.VMEM((SQ, 1024), jnp.float32),
            pltpu.VMEM((2, 128, 1024), jnp.bfloat16),
            pltpu.VMEM((2, 128, 1024), jnp.bfloat16),
            pltpu.VMEM((128, 1024), jnp.bfloat16),
            pltpu.VMEM((2, 128, 1024), jnp.bfloat16),
            pltpu.SemaphoreType.DMA((N_DEV,)),
            pltpu.SemaphoreType.DMA((N_DEV,)),
            pltpu.SemaphoreType.DMA((16,)),
            pltpu.SemaphoreType.DMA((4,)),
            pltpu.SemaphoreType.DMA((4,)),
            pltpu.SemaphoreType.DMA((2,)),
            pltpu.SemaphoreType.DMA((2,)),
            pltpu.SemaphoreType.DMA((2,)),
            pltpu.SemaphoreType.REGULAR,
            pltpu.SemaphoreType.REGULAR,
        ],
        compiler_params=pltpu.CompilerParams(
            collective_id=0, vmem_limit_bytes=60 * 2**20),
    )(x, Wq, K_ext, V_ext, Wo)
